# Initial kernel scaffold; baseline (speedup 1.0000x reference)
import functools

import jax
import jax.numpy as jnp
from jax import lax
from jax.experimental import pallas as pl
from jax.experimental.pallas import tpu as pltpu
from jax.experimental.pallas import tpu_sc as plsc

_NC = 2
_NS = 16
_NW = _NC * _NS
_CH = 80
_B = 64


def _sc_mesh():
    return plsc.VectorSubcoreMesh(core_axis_name="c", subcore_axis_name="s")


def _sc_ea_pass(dst3, edge_attr, n_nodes):
    nw, nch, ch = dst3.shape
    e, de = edge_attr.shape
    rps = n_nodes // _NS

    @functools.partial(
        pl.kernel,
        out_type=jax.ShapeDtypeStruct((_NC, n_nodes, de), jnp.float32),
        mesh=_sc_mesh(),
        scratch_types=[
            pltpu.VMEM((nch, ch), jnp.int32),
            pltpu.VMEM((ch, de), jnp.float32),
            pltpu.VMEM((rps, de), jnp.float32),
            pltpu.VMEM_SHARED((n_nodes, de), jnp.float32),
        ],
    )
    def k(dst_hbm, ea_hbm, out_hbm, didx_v, ea_v, z_v, acc_sh):
        cid = lax.axis_index("c")
        sid = lax.axis_index("s")
        wid = sid * _NC + cid

        @pl.loop(0, rps)
        def _zr(r):
            @pl.loop(0, de, step=16)
            def _zc(c):
                z_v[r, pl.ds(c, 16)] = jnp.zeros((16,), jnp.float32)

        pltpu.sync_copy(z_v, acc_sh.at[pl.ds(sid * rps, rps)])
        plsc.subcore_barrier()

        pltpu.sync_copy(dst_hbm.at[wid], didx_v)
        base_e = wid * (nch * ch)

        @pl.loop(0, nch)
        def _acc(j):
            pltpu.sync_copy(ea_hbm.at[pl.ds(base_e + j * ch, ch)], ea_v)
            pltpu.sync_copy(ea_v, acc_sh.at[didx_v.at[j]], add=True)

        plsc.subcore_barrier()
        pltpu.sync_copy(acc_sh.at[pl.ds(sid * rps, rps)],
                        out_hbm.at[cid, pl.ds(sid * rps, rps)])

    return k(dst3, edge_attr)


def _sc_gather_pass(table, src3, dst3):
    n_nodes, d = table.shape
    nw, nch, ch = src3.shape
    rps = n_nodes // _NS
    zrows = 125

    @functools.partial(
        pl.kernel,
        out_type=jax.ShapeDtypeStruct((_NC, n_nodes, d), jnp.float32),
        mesh=_sc_mesh(),
        scratch_types=[
            pltpu.VMEM((nch, ch), jnp.int32),
            pltpu.VMEM((nch, ch), jnp.int32),
            pltpu.VMEM((ch, d), jnp.float32),
            pltpu.VMEM((zrows, d), jnp.float32),
            pltpu.VMEM_SHARED((n_nodes, d), jnp.float32),
            pltpu.SemaphoreType.DMA,
        ],
    )
    def k(tbl_hbm, src_hbm, dst_hbm, out_hbm,
          sidx_v, didx_v, rows_v, z_v, acc_sh, sem):
        cid = lax.axis_index("c")
        sid = lax.axis_index("s")
        wid = sid * _NC + cid

        @pl.loop(0, zrows)
        def _zr(r):
            @pl.loop(0, d, step=16)
            def _zc(c):
                z_v[r, pl.ds(c, 16)] = jnp.zeros((16,), jnp.float32)

        @pl.loop(0, rps // zrows)
        def _zt(t):
            pltpu.sync_copy(z_v, acc_sh.at[pl.ds(sid * rps + t * zrows, zrows)])

        plsc.subcore_barrier()

        pltpu.sync_copy(src_hbm.at[wid], sidx_v)
        pltpu.sync_copy(dst_hbm.at[wid], didx_v)

        @pl.loop(0, nch)
        def _acc(j):
            pltpu.async_copy(tbl_hbm.at[sidx_v.at[j]], rows_v, sem).wait()
            pltpu.sync_copy(rows_v, acc_sh.at[didx_v.at[j]], add=True)

        plsc.subcore_barrier()
        pltpu.sync_copy(acc_sh.at[pl.ds(sid * rps, rps)],
                        out_hbm.at[cid, pl.ds(sid * rps, rps)])

    return k(table, src3, dst3)


def _tc_linear(x, w, b, rb):
    n, d = x.shape
    kdim = w.shape[1]

    def body(x_ref, w_ref, b_ref, o_ref):
        o_ref[...] = (
            jnp.dot(x_ref[...], w_ref[...], preferred_element_type=jnp.float32)
            + b_ref[...]
        )

    return pl.pallas_call(
        body,
        grid=(n // rb,),
        in_specs=[
            pl.BlockSpec((rb, d), lambda i: (i, 0)),
            pl.BlockSpec((d, kdim), lambda i: (0, 0)),
            pl.BlockSpec((1, kdim), lambda i: (0, 0)),
        ],
        out_specs=pl.BlockSpec((rb, kdim), lambda i: (i, 0)),
        out_shape=jax.ShapeDtypeStruct((n, kdim), jnp.float32),
    )(x, w, b)


def _tc_mid(p1, ea, we1, w2, b2, rb):
    _, n, d = p1.shape
    de = ea.shape[2]

    def body(p_ref, ea_ref, we1_ref, w2_ref, b2_ref, o_ref):
        agg = p_ref[0] + p_ref[1] + jnp.dot(
            ea_ref[0] + ea_ref[1], we1_ref[...],
            preferred_element_type=jnp.float32)
        h = jnp.maximum(agg, 0.0)
        o_ref[...] = (
            jnp.dot(h, w2_ref[...], preferred_element_type=jnp.float32)
            + b2_ref[...]
        )

    return pl.pallas_call(
        body,
        grid=(n // rb,),
        in_specs=[
            pl.BlockSpec((2, rb, d), lambda i: (0, i, 0)),
            pl.BlockSpec((2, rb, de), lambda i: (0, i, 0)),
            pl.BlockSpec((de, d), lambda i: (0, 0)),
            pl.BlockSpec((d, d), lambda i: (0, 0)),
            pl.BlockSpec((1, d), lambda i: (0, 0)),
        ],
        out_specs=pl.BlockSpec((rb, d), lambda i: (i, 0)),
        out_shape=jax.ShapeDtypeStruct((n, d), jnp.float32),
    )(p1, ea, we1, w2, b2)


def _tc_final(p2, ea, we2, batch3, wlin, blin, nb):
    _, n, d = p2.shape
    de = ea.shape[2]
    c = wlin.shape[1]
    rb = n // nb

    def body(p_ref, ea_ref, we2_ref, b_ref, wlin_ref, blin_ref, o_ref, acc):
        i = pl.program_id(0)

        @pl.when(i == 0)
        def _init():
            acc[...] = jnp.zeros_like(acc)

        hf = p_ref[0] + p_ref[1] + jnp.dot(
            ea_ref[0] + ea_ref[1], we2_ref[...],
            preferred_element_type=jnp.float32)
        bv = b_ref[0, 0, :]
        onehot = (bv[None, :] ==
                  lax.broadcasted_iota(jnp.int32, (_B, rb), 0)
                  ).astype(jnp.float32)
        acc[...] += jnp.dot(onehot, hf, preferred_element_type=jnp.float32)

        @pl.when(i == nb - 1)
        def _fin():
            o_ref[...] = (
                jnp.dot(acc[...], wlin_ref[...],
                        preferred_element_type=jnp.float32)
                + blin_ref[...]
            )

    return pl.pallas_call(
        body,
        grid=(nb,),
        in_specs=[
            pl.BlockSpec((2, rb, d), lambda i: (0, i, 0)),
            pl.BlockSpec((2, rb, de), lambda i: (0, i, 0)),
            pl.BlockSpec((de, d), lambda i: (0, 0)),
            pl.BlockSpec((1, 1, rb), lambda i: (i, 0, 0)),
            pl.BlockSpec((d, c), lambda i: (0, 0)),
            pl.BlockSpec((1, c), lambda i: (0, 0)),
        ],
        out_specs=pl.BlockSpec((_B, c), lambda i: (0, 0)),
        out_shape=jax.ShapeDtypeStruct((_B, c), jnp.float32),
        scratch_shapes=[pltpu.VMEM((_B, d), jnp.float32)],
    )(p2, ea, we2, batch3, wlin, blin)


def kernel(x, edge_index, edge_attr, batch, W1, b1, We1, W2, b2, We2,
           Wlin, blin):
    n, d = x.shape
    nb = 10

    src3 = edge_index[0].reshape(_NW, -1, _CH)
    dst3 = edge_index[1].reshape(_NW, -1, _CH)

    ea_p = _sc_ea_pass(dst3, edge_attr, n)
    h1 = _tc_linear(x, W1, b1.reshape(1, -1), n // 5)
    p1 = _sc_gather_pass(h1, src3, dst3)
    h2 = _tc_mid(p1, ea_p, We1, W2, b2.reshape(1, -1), n // 5)
    p2 = _sc_gather_pass(h2, src3, dst3)
    return _tc_final(p2, ea_p, We2, batch.reshape(nb, 1, -1),
                     Wlin, blin.reshape(1, -1), nb)


# speedup vs baseline: 5.3412x; 5.3412x over previous
import functools

import jax
import jax.numpy as jnp
from jax import lax
from jax.experimental import pallas as pl
from jax.experimental.pallas import tpu as pltpu
from jax.experimental.pallas import tpu_sc as plsc

_NC = 2
_NS = 16
_NW = _NC * _NS
_CH = 80
_B = 64


def _sc_mesh():
    return plsc.VectorSubcoreMesh(core_axis_name="c", subcore_axis_name="s")


def _sc_ea_pass(dst3, ea4, n_pad, de):
    nw, nch, ch = dst3.shape
    per_row = 128 // de
    nrow = ch // per_row
    rps = n_pad // _NS

    @functools.partial(
        pl.kernel,
        out_type=jax.ShapeDtypeStruct((_NC, n_pad, 128), jnp.float32),
        mesh=_sc_mesh(),
        scratch_types=[
            pltpu.VMEM((nch, ch), jnp.int32),
            pltpu.VMEM((nrow, 128), jnp.float32),
            pltpu.VMEM((ch, 128), jnp.float32),
            pltpu.VMEM_SHARED((n_pad, 128), jnp.float32),
        ],
    )
    def k(dst_hbm, ea_hbm, out_hbm, didx_v, pk_v, ea_v, acc_sh):
        cid = lax.axis_index("c")
        sid = lax.axis_index("s")
        wid = sid * _NC + cid

        @pl.loop(0, ch)
        def _zr(r):
            @pl.loop(0, 128, step=16)
            def _zc(c):
                ea_v[r, pl.ds(c, 16)] = jnp.zeros((16,), jnp.float32)

        @pl.loop(0, rps // ch)
        def _zt(t):
            pltpu.sync_copy(ea_v, acc_sh.at[pl.ds(sid * rps + t * ch, ch)])

        plsc.subcore_barrier()

        pltpu.sync_copy(dst_hbm.at[wid], didx_v)

        @pl.loop(0, nch)
        def _acc(j):
            pltpu.sync_copy(ea_hbm.at[wid, j], pk_v)

            @pl.loop(0, nrow)
            def _spread(q):
                for u in range(per_row):
                    ea_v[q * per_row + u, pl.ds(0, de)] = \
                        pk_v[q, pl.ds(u * de, de)]

            pltpu.sync_copy(ea_v, acc_sh.at[didx_v.at[j]], add=True)

        plsc.subcore_barrier()
        pltpu.sync_copy(acc_sh.at[pl.ds(sid * rps, rps)],
                        out_hbm.at[cid, pl.ds(sid * rps, rps)])

    return k(dst3, ea4)


def _sc_gather_pass(table, src3, dst3, n_pad):
    n_nodes, d = table.shape
    nw, nch, ch = src3.shape
    rps = n_pad // _NS

    @functools.partial(
        pl.kernel,
        out_type=jax.ShapeDtypeStruct((_NC, n_pad, d), jnp.float32),
        mesh=_sc_mesh(),
        scratch_types=[
            pltpu.VMEM((nch, ch), jnp.int32),
            pltpu.VMEM((nch, ch), jnp.int32),
            pltpu.VMEM((ch, d), jnp.float32),
            pltpu.VMEM_SHARED((n_pad, d), jnp.float32),
            pltpu.SemaphoreType.DMA,
        ],
    )
    def k(tbl_hbm, src_hbm, dst_hbm, out_hbm,
          sidx_v, didx_v, rows_v, acc_sh, sem):
        cid = lax.axis_index("c")
        sid = lax.axis_index("s")
        wid = sid * _NC + cid

        @pl.loop(0, ch)
        def _zr(r):
            @pl.loop(0, d, step=16)
            def _zc(c):
                rows_v[r, pl.ds(c, 16)] = jnp.zeros((16,), jnp.float32)

        @pl.loop(0, rps // ch)
        def _zt(t):
            pltpu.sync_copy(rows_v, acc_sh.at[pl.ds(sid * rps + t * ch, ch)])

        plsc.subcore_barrier()

        pltpu.sync_copy(src_hbm.at[wid], sidx_v)
        pltpu.sync_copy(dst_hbm.at[wid], didx_v)

        @pl.loop(0, nch)
        def _acc(j):
            pltpu.async_copy(tbl_hbm.at[sidx_v.at[j]], rows_v, sem).wait()
            pltpu.sync_copy(rows_v, acc_sh.at[didx_v.at[j]], add=True)

        plsc.subcore_barrier()
        pltpu.sync_copy(acc_sh.at[pl.ds(sid * rps, rps)],
                        out_hbm.at[cid, pl.ds(sid * rps, rps)])

    return k(table, src3, dst3)


def _tc_linear(x, w, b, rb):
    n, d = x.shape
    kdim = w.shape[1]

    def body(x_ref, w_ref, b_ref, o_ref):
        o_ref[...] = (
            jnp.dot(x_ref[...], w_ref[...], preferred_element_type=jnp.float32)
            + b_ref[...]
        )

    return pl.pallas_call(
        body,
        grid=(n // rb,),
        in_specs=[
            pl.BlockSpec((rb, d), lambda i: (i, 0)),
            pl.BlockSpec((d, kdim), lambda i: (0, 0)),
            pl.BlockSpec((1, kdim), lambda i: (0, 0)),
        ],
        out_specs=pl.BlockSpec((rb, kdim), lambda i: (i, 0)),
        out_shape=jax.ShapeDtypeStruct((n, kdim), jnp.float32),
    )(x, w, b)


def _tc_mid(p1, ea, we1, w2, b2, n, rb):
    d = p1.shape[2]
    de = we1.shape[0]

    def body(p_ref, ea_ref, we1_ref, w2_ref, b2_ref, o_ref):
        agg = p_ref[0] + p_ref[1] + jnp.dot(
            ea_ref[0, :, :de] + ea_ref[1, :, :de], we1_ref[...],
            preferred_element_type=jnp.float32)
        h = jnp.maximum(agg, 0.0)
        o_ref[...] = (
            jnp.dot(h, w2_ref[...], preferred_element_type=jnp.float32)
            + b2_ref[...]
        )

    return pl.pallas_call(
        body,
        grid=(n // rb,),
        in_specs=[
            pl.BlockSpec((2, rb, d), lambda i: (0, i, 0)),
            pl.BlockSpec((2, rb, 128), lambda i: (0, i, 0)),
            pl.BlockSpec((de, d), lambda i: (0, 0)),
            pl.BlockSpec((d, d), lambda i: (0, 0)),
            pl.BlockSpec((1, d), lambda i: (0, 0)),
        ],
        out_specs=pl.BlockSpec((rb, d), lambda i: (i, 0)),
        out_shape=jax.ShapeDtypeStruct((n, d), jnp.float32),
    )(p1, ea, we1, w2, b2)


def _tc_final(p2, ea, we2, batch3, wlin, blin, n, nb):
    d = p2.shape[2]
    de = we2.shape[0]
    c = wlin.shape[1]
    rb = n // nb

    def body(p_ref, ea_ref, we2_ref, b_ref, wlin_ref, blin_ref, o_ref, acc):
        i = pl.program_id(0)

        @pl.when(i == 0)
        def _init():
            acc[...] = jnp.zeros_like(acc)

        hf = p_ref[0] + p_ref[1] + jnp.dot(
            ea_ref[0, :, :de] + ea_ref[1, :, :de], we2_ref[...],
            preferred_element_type=jnp.float32)
        bv = b_ref[0, 0, :]
        onehot = (bv[None, :] ==
                  lax.broadcasted_iota(jnp.int32, (_B, rb), 0)
                  ).astype(jnp.float32)
        acc[...] += jnp.dot(onehot, hf, preferred_element_type=jnp.float32)

        @pl.when(i == nb - 1)
        def _fin():
            o_ref[...] = (
                jnp.dot(acc[...], wlin_ref[...],
                        preferred_element_type=jnp.float32)
                + blin_ref[...]
            )

    return pl.pallas_call(
        body,
        grid=(nb,),
        in_specs=[
            pl.BlockSpec((2, rb, d), lambda i: (0, i, 0)),
            pl.BlockSpec((2, rb, 128), lambda i: (0, i, 0)),
            pl.BlockSpec((de, d), lambda i: (0, 0)),
            pl.BlockSpec((1, 1, rb), lambda i: (i, 0, 0)),
            pl.BlockSpec((d, c), lambda i: (0, 0)),
            pl.BlockSpec((1, c), lambda i: (0, 0)),
        ],
        out_specs=pl.BlockSpec((_B, c), lambda i: (0, 0)),
        out_shape=jax.ShapeDtypeStruct((_B, c), jnp.float32),
        scratch_shapes=[pltpu.VMEM((_B, d), jnp.float32)],
    )(p2, ea, we2, batch3, wlin, blin)


def kernel(x, edge_index, edge_attr, batch, W1, b1, We1, W2, b2, We2,
           Wlin, blin):
    n, d = x.shape
    nb = 10
    n_pad = 10240

    src3 = edge_index[0].reshape(_NW, -1, _CH)
    dst3 = edge_index[1].reshape(_NW, -1, _CH)
    de = edge_attr.shape[1]
    nch = dst3.shape[1]
    ea4 = edge_attr.reshape(_NW, nch, (_CH * de) // 128, 128)

    ea_p = _sc_ea_pass(dst3, ea4, n_pad, de)
    h1 = _tc_linear(x, W1, b1.reshape(1, -1), n // 5)
    p1 = _sc_gather_pass(h1, src3, dst3, n_pad)
    h2 = _tc_mid(p1, ea_p, We1, W2, b2.reshape(1, -1), n, n // 5)
    p2 = _sc_gather_pass(h2, src3, dst3, n_pad)
    return _tc_final(p2, ea_p, We2, batch.reshape(nb, 1, -1),
                     Wlin, blin.reshape(1, -1), n, nb)


# speedup vs baseline: 7.1909x; 1.3463x over previous
import functools

import jax
import jax.numpy as jnp
from jax import lax
from jax.experimental import pallas as pl
from jax.experimental.pallas import tpu as pltpu
from jax.experimental.pallas import tpu_sc as plsc

_NC = 2
_NS = 16
_NW = _NC * _NS
_CH = 80
_B = 64


def _sc_mesh():
    return plsc.VectorSubcoreMesh(core_axis_name="c", subcore_axis_name="s")


def _sc_ea_pass(ei5, ea4, n_pad, de):
    _, nw, nseg, seg, ch = ei5.shape
    nch = nseg * seg
    per_row = 128 // de
    nrow = ch // per_row
    rps = n_pad // _NS

    @functools.partial(
        pl.kernel,
        out_type=jax.ShapeDtypeStruct((_NC, n_pad, 128), jnp.float32),
        mesh=_sc_mesh(),
        scratch_types=[
            pltpu.VMEM((nseg, seg, ch), jnp.int32),
            pltpu.VMEM((nrow, 128), jnp.float32),
            pltpu.VMEM((ch, 128), jnp.float32),
            pltpu.VMEM_SHARED((n_pad, 128), jnp.float32),
        ],
    )
    def k(ei_hbm, ea_hbm, out_hbm, didx_v, pk_v, ea_v, acc_sh):
        cid = lax.axis_index("c")
        sid = lax.axis_index("s")
        wid = sid * _NC + cid

        @pl.loop(0, ch)
        def _zr(r):
            @pl.loop(0, 128, step=16)
            def _zc(c):
                ea_v[r, pl.ds(c, 16)] = jnp.zeros((16,), jnp.float32)

        @pl.loop(0, rps // ch)
        def _zt(t):
            pltpu.sync_copy(ea_v, acc_sh.at[pl.ds(sid * rps + t * ch, ch)])

        plsc.subcore_barrier()

        pltpu.sync_copy(ei_hbm.at[1, wid], didx_v)

        @pl.loop(0, nch)
        def _acc(j):
            pltpu.sync_copy(ea_hbm.at[wid, j], pk_v)

            @pl.loop(0, nrow)
            def _spread(q):
                for u in range(per_row):
                    ea_v[q * per_row + u, pl.ds(0, de)] = \
                        pk_v[q, pl.ds(u * de, de)]

            pltpu.sync_copy(
                ea_v, acc_sh.at[didx_v.at[lax.div(j, seg), lax.rem(j, seg)]],
                add=True)

        plsc.subcore_barrier()
        pltpu.sync_copy(acc_sh.at[pl.ds(sid * rps, rps)],
                        out_hbm.at[cid, pl.ds(sid * rps, rps)])

    return k(ei5, ea4)


def _sc_gather_pass(table, ei5, n_pad):
    n_nodes, d = table.shape
    _, nw, nseg, seg, ch = ei5.shape
    nch = nseg * seg
    rps = n_pad // _NS
    assert nch % 2 == 1

    @functools.partial(
        pl.kernel,
        out_type=jax.ShapeDtypeStruct((_NC, n_pad, d), jnp.float32),
        mesh=_sc_mesh(),
        scratch_types=[
            pltpu.VMEM((nseg, seg, ch), jnp.int32),
            pltpu.VMEM((seg, ch), jnp.int32),
            pltpu.VMEM((ch, d), jnp.float32),
            pltpu.VMEM((ch, d), jnp.float32),
            pltpu.VMEM_SHARED((n_pad, d), jnp.float32),
            pltpu.SemaphoreType.DMA,
            pltpu.SemaphoreType.DMA,
        ],
    )
    def k(tbl_hbm, ei_hbm, out_hbm,
          sidx_v, didx_v, rows_a, rows_b, acc_sh, sem_a, sem_b):
        cid = lax.axis_index("c")
        sid = lax.axis_index("s")
        wid = sid * _NC + cid

        @pl.loop(0, ch)
        def _zr(r):
            @pl.loop(0, d, step=16)
            def _zc(c):
                rows_a[r, pl.ds(c, 16)] = jnp.zeros((16,), jnp.float32)

        @pl.loop(0, rps // ch)
        def _zt(t):
            pltpu.sync_copy(rows_a, acc_sh.at[pl.ds(sid * rps + t * ch, ch)])

        plsc.subcore_barrier()

        pltpu.sync_copy(ei_hbm.at[0, wid], sidx_v)

        def start_gather(j, buf, sem):
            pltpu.async_copy(
                tbl_hbm.at[sidx_v.at[lax.div(j, seg), lax.rem(j, seg)]],
                buf, sem)

        def wait_gather(buf, sem):
            pltpu.make_async_copy(tbl_hbm.at[sidx_v.at[0, 0]], buf, sem).wait()

        def scatter_step(j, buf):
            @pl.when(lax.rem(j, seg) == 0)
            def _seg():
                pltpu.sync_copy(ei_hbm.at[1, wid, lax.div(j, seg)], didx_v)

            pltpu.sync_copy(buf, acc_sh.at[didx_v.at[lax.rem(j, seg)]],
                            add=True)

        start_gather(0, rows_a, sem_a)

        @pl.loop(0, (nch - 1) // 2)
        def _ring(t):
            j = 2 * t
            start_gather(j + 1, rows_b, sem_b)
            wait_gather(rows_a, sem_a)
            scatter_step(j, rows_a)
            start_gather(j + 2, rows_a, sem_a)
            wait_gather(rows_b, sem_b)
            scatter_step(j + 1, rows_b)

        wait_gather(rows_a, sem_a)
        scatter_step(nch - 1, rows_a)

        plsc.subcore_barrier()
        pltpu.sync_copy(acc_sh.at[pl.ds(sid * rps, rps)],
                        out_hbm.at[cid, pl.ds(sid * rps, rps)])

    return k(table, ei5)


def _tc_linear(x, w, b, rb):
    n, d = x.shape
    kdim = w.shape[1]

    def body(x_ref, w_ref, b_ref, o_ref):
        o_ref[...] = (
            jnp.dot(x_ref[...], w_ref[...], preferred_element_type=jnp.float32)
            + b_ref[...]
        )

    return pl.pallas_call(
        body,
        grid=(n // rb,),
        in_specs=[
            pl.BlockSpec((rb, d), lambda i: (i, 0)),
            pl.BlockSpec((d, kdim), lambda i: (0, 0)),
            pl.BlockSpec((1, kdim), lambda i: (0, 0)),
        ],
        out_specs=pl.BlockSpec((rb, kdim), lambda i: (i, 0)),
        out_shape=jax.ShapeDtypeStruct((n, kdim), jnp.float32),
    )(x, w, b)


def _tc_mid(p1, ea, we1, w2, b2, n, rb):
    d = p1.shape[2]
    de = we1.shape[0]

    def body(p_ref, ea_ref, we1_ref, w2_ref, b2_ref, o_ref):
        agg = p_ref[0] + p_ref[1] + jnp.dot(
            ea_ref[0, :, :de] + ea_ref[1, :, :de], we1_ref[...],
            preferred_element_type=jnp.float32)
        h = jnp.maximum(agg, 0.0)
        o_ref[...] = (
            jnp.dot(h, w2_ref[...], preferred_element_type=jnp.float32)
            + b2_ref[...]
        )

    return pl.pallas_call(
        body,
        grid=(n // rb,),
        in_specs=[
            pl.BlockSpec((2, rb, d), lambda i: (0, i, 0)),
            pl.BlockSpec((2, rb, 128), lambda i: (0, i, 0)),
            pl.BlockSpec((de, d), lambda i: (0, 0)),
            pl.BlockSpec((d, d), lambda i: (0, 0)),
            pl.BlockSpec((1, d), lambda i: (0, 0)),
        ],
        out_specs=pl.BlockSpec((rb, d), lambda i: (i, 0)),
        out_shape=jax.ShapeDtypeStruct((n, d), jnp.float32),
    )(p1, ea, we1, w2, b2)


def _tc_final(p2, ea, we2, batch3, wlin, blin, n, nb):
    d = p2.shape[2]
    de = we2.shape[0]
    c = wlin.shape[1]
    rb = n // nb

    def body(p_ref, ea_ref, we2_ref, b_ref, wlin_ref, blin_ref, o_ref, acc):
        i = pl.program_id(0)

        @pl.when(i == 0)
        def _init():
            acc[...] = jnp.zeros_like(acc)

        hf = p_ref[0] + p_ref[1] + jnp.dot(
            ea_ref[0, :, :de] + ea_ref[1, :, :de], we2_ref[...],
            preferred_element_type=jnp.float32)
        bv = b_ref[0, 0, :]
        onehot = (bv[None, :] ==
                  lax.broadcasted_iota(jnp.int32, (_B, rb), 0)
                  ).astype(jnp.float32)
        acc[...] += jnp.dot(onehot, hf, preferred_element_type=jnp.float32)

        @pl.when(i == nb - 1)
        def _fin():
            o_ref[...] = (
                jnp.dot(acc[...], wlin_ref[...],
                        preferred_element_type=jnp.float32)
                + blin_ref[...]
            )

    return pl.pallas_call(
        body,
        grid=(nb,),
        in_specs=[
            pl.BlockSpec((2, rb, d), lambda i: (0, i, 0)),
            pl.BlockSpec((2, rb, 128), lambda i: (0, i, 0)),
            pl.BlockSpec((de, d), lambda i: (0, 0)),
            pl.BlockSpec((1, 1, rb), lambda i: (i, 0, 0)),
            pl.BlockSpec((d, c), lambda i: (0, 0)),
            pl.BlockSpec((1, c), lambda i: (0, 0)),
        ],
        out_specs=pl.BlockSpec((_B, c), lambda i: (0, 0)),
        out_shape=jax.ShapeDtypeStruct((_B, c), jnp.float32),
        scratch_shapes=[pltpu.VMEM((_B, d), jnp.float32)],
    )(p2, ea, we2, batch3, wlin, blin)


def kernel(x, edge_index, edge_attr, batch, W1, b1, We1, W2, b2, We2,
           Wlin, blin):
    n, d = x.shape
    nb = 10
    n_pad = 10240

    e = edge_index.shape[1]
    de = edge_attr.shape[1]
    nseg, seg = 5, 25
    nch = e // (_NW * _CH)
    ei5 = edge_index.reshape(2, _NW, nseg, seg, _CH)
    ea4 = edge_attr.reshape(_NW, nch, (_CH * de) // 128, 128)

    ea_p = _sc_ea_pass(ei5, ea4, n_pad, de)
    h1 = _tc_linear(x, W1, b1.reshape(1, -1), n // 5)
    p1 = _sc_gather_pass(h1, ei5, n_pad)
    h2 = _tc_mid(p1, ea_p, We1, W2, b2.reshape(1, -1), n, n // 5)
    p2 = _sc_gather_pass(h2, ei5, n_pad)
    return _tc_final(p2, ea_p, We2, batch.reshape(nb, 1, -1),
                     Wlin, blin.reshape(1, -1), n, nb)


# speedup vs baseline: 8.4949x; 1.1813x over previous
import functools

import jax
import jax.numpy as jnp
from jax import lax
from jax.experimental import pallas as pl
from jax.experimental.pallas import tpu as pltpu
from jax.experimental.pallas import tpu_sc as plsc

_NC = 2
_NS = 16
_NW = _NC * _NS
_CH = 80
_B = 64


def _sc_mesh():
    return plsc.VectorSubcoreMesh(core_axis_name="c", subcore_axis_name="s")


def _sc_ea_pass(ei5, ea4, n_pad, de):
    _, nw, nseg, seg, ch = ei5.shape
    nch = nseg * seg
    per_row = 128 // de
    nrow = ch // per_row
    rps = n_pad // _NS

    assert nch % 2 == 1

    @functools.partial(
        pl.kernel,
        out_type=jax.ShapeDtypeStruct((_NC, n_pad, 128), jnp.float32),
        mesh=_sc_mesh(),
        scratch_types=[
            pltpu.VMEM((nseg, seg, ch), jnp.int32),
            pltpu.VMEM((nrow, 128), jnp.float32),
            pltpu.VMEM((nrow, 128), jnp.float32),
            pltpu.VMEM((ch, 128), jnp.float32),
            pltpu.VMEM((ch, 128), jnp.float32),
            pltpu.VMEM_SHARED((n_pad, 128), jnp.float32),
            pltpu.SemaphoreType.DMA,
            pltpu.SemaphoreType.DMA,
            pltpu.SemaphoreType.DMA,
            pltpu.SemaphoreType.DMA,
        ],
    )
    def k(ei_hbm, ea_hbm, out_hbm, didx_v, pk_a, pk_b, ea_a, ea_b, acc_sh,
          sem_la, sem_lb, sem_sa, sem_sb):
        cid = lax.axis_index("c")
        sid = lax.axis_index("s")
        wid = sid * _NC + cid

        for buf in (ea_a, ea_b):
            @pl.loop(0, ch)
            def _zr(r, buf=buf):
                @pl.loop(0, 128, step=16)
                def _zc(c):
                    buf[r, pl.ds(c, 16)] = jnp.zeros((16,), jnp.float32)

        @pl.loop(0, rps // ch)
        def _zt(t):
            pltpu.sync_copy(ea_a, acc_sh.at[pl.ds(sid * rps + t * ch, ch)])

        plsc.subcore_barrier()

        pltpu.sync_copy(ei_hbm.at[1, wid], didx_v)

        def start_load(j, pk, sem):
            pltpu.async_copy(ea_hbm.at[wid, j], pk, sem)

        def wait_load(pk, sem):
            pltpu.make_async_copy(ea_hbm.at[wid, 0], pk, sem).wait()

        def spread(pk, ea_v):
            @pl.loop(0, nrow)
            def _spread(q):
                for u in range(per_row):
                    ea_v[q * per_row + u, pl.ds(0, de)] = \
                        pk[q, pl.ds(u * de, de)]

        def start_scatter(j, ea_v, sem):
            pltpu.async_copy(
                ea_v, acc_sh.at[didx_v.at[lax.div(j, seg), lax.rem(j, seg)]],
                sem, add=True)

        def wait_scatter(ea_v, sem):
            pltpu.make_async_copy(ea_v, acc_sh.at[didx_v.at[0, 0]], sem).wait()

        def pair(j, first):
            if not first:
                wait_scatter(ea_a, sem_sa)
            wait_load(pk_a, sem_la)
            spread(pk_a, ea_a)
            start_scatter(j, ea_a, sem_sa)

            @pl.when(j + 2 < nch)
            def _la():
                start_load(j + 2, pk_a, sem_la)

            if not first:
                wait_scatter(ea_b, sem_sb)
            wait_load(pk_b, sem_lb)
            spread(pk_b, ea_b)
            start_scatter(j + 1, ea_b, sem_sb)

            @pl.when(j + 3 < nch)
            def _lb():
                start_load(j + 3, pk_b, sem_lb)

        start_load(0, pk_a, sem_la)
        start_load(1, pk_b, sem_lb)
        pair(0, True)

        @pl.loop(1, (nch - 1) // 2)
        def _ring(t):
            pair(2 * t, False)

        wait_scatter(ea_a, sem_sa)
        wait_load(pk_a, sem_la)
        spread(pk_a, ea_a)
        start_scatter(nch - 1, ea_a, sem_sa)
        wait_scatter(ea_a, sem_sa)
        wait_scatter(ea_b, sem_sb)

        plsc.subcore_barrier()
        pltpu.sync_copy(acc_sh.at[pl.ds(sid * rps, rps)],
                        out_hbm.at[cid, pl.ds(sid * rps, rps)])

    return k(ei5, ea4)


def _sc_gather_pass(table, ei5, n_pad):
    n_nodes, d = table.shape
    _, nw, nseg, seg, ch = ei5.shape
    nch = nseg * seg
    rps = n_pad // _NS
    assert nch % 2 == 1

    @functools.partial(
        pl.kernel,
        out_type=jax.ShapeDtypeStruct((_NC, n_pad, d), jnp.float32),
        mesh=_sc_mesh(),
        scratch_types=[
            pltpu.VMEM((nseg, seg, ch), jnp.int32),
            pltpu.VMEM((seg, ch), jnp.int32),
            pltpu.VMEM((ch, d), jnp.float32),
            pltpu.VMEM((ch, d), jnp.float32),
            pltpu.VMEM_SHARED((n_pad, d), jnp.float32),
            pltpu.SemaphoreType.DMA,
            pltpu.SemaphoreType.DMA,
        ],
    )
    def k(tbl_hbm, ei_hbm, out_hbm,
          sidx_v, didx_v, rows_a, rows_b, acc_sh, sem_a, sem_b):
        cid = lax.axis_index("c")
        sid = lax.axis_index("s")
        wid = sid * _NC + cid

        @pl.loop(0, ch)
        def _zr(r):
            @pl.loop(0, d, step=16)
            def _zc(c):
                rows_a[r, pl.ds(c, 16)] = jnp.zeros((16,), jnp.float32)

        @pl.loop(0, rps // ch)
        def _zt(t):
            pltpu.sync_copy(rows_a, acc_sh.at[pl.ds(sid * rps + t * ch, ch)])

        plsc.subcore_barrier()

        pltpu.sync_copy(ei_hbm.at[0, wid], sidx_v)

        def start_gather(j, buf, sem):
            pltpu.async_copy(
                tbl_hbm.at[sidx_v.at[lax.div(j, seg), lax.rem(j, seg)]],
                buf, sem)

        def wait_gather(buf, sem):
            pltpu.make_async_copy(tbl_hbm.at[sidx_v.at[0, 0]], buf, sem).wait()

        def scatter_step(j, buf):
            @pl.when(lax.rem(j, seg) == 0)
            def _seg():
                pltpu.sync_copy(ei_hbm.at[1, wid, lax.div(j, seg)], didx_v)

            pltpu.sync_copy(buf, acc_sh.at[didx_v.at[lax.rem(j, seg)]],
                            add=True)

        start_gather(0, rows_a, sem_a)

        @pl.loop(0, (nch - 1) // 2)
        def _ring(t):
            j = 2 * t
            start_gather(j + 1, rows_b, sem_b)
            wait_gather(rows_a, sem_a)
            scatter_step(j, rows_a)
            start_gather(j + 2, rows_a, sem_a)
            wait_gather(rows_b, sem_b)
            scatter_step(j + 1, rows_b)

        wait_gather(rows_a, sem_a)
        scatter_step(nch - 1, rows_a)

        plsc.subcore_barrier()
        pltpu.sync_copy(acc_sh.at[pl.ds(sid * rps, rps)],
                        out_hbm.at[cid, pl.ds(sid * rps, rps)])

    return k(table, ei5)


def _tc_linear(x, w, b, rb):
    n, d = x.shape
    kdim = w.shape[1]

    def body(x_ref, w_ref, b_ref, o_ref):
        o_ref[...] = (
            jnp.dot(x_ref[...], w_ref[...], preferred_element_type=jnp.float32)
            + b_ref[...]
        )

    return pl.pallas_call(
        body,
        grid=(n // rb,),
        in_specs=[
            pl.BlockSpec((rb, d), lambda i: (i, 0)),
            pl.BlockSpec((d, kdim), lambda i: (0, 0)),
            pl.BlockSpec((1, kdim), lambda i: (0, 0)),
        ],
        out_specs=pl.BlockSpec((rb, kdim), lambda i: (i, 0)),
        out_shape=jax.ShapeDtypeStruct((n, kdim), jnp.float32),
    )(x, w, b)


def _tc_mid(p1, ea, we1, w2, b2, n, rb):
    d = p1.shape[2]
    de = we1.shape[0]

    def body(p_ref, ea_ref, we1_ref, w2_ref, b2_ref, o_ref):
        agg = p_ref[0] + p_ref[1] + jnp.dot(
            ea_ref[0, :, :de] + ea_ref[1, :, :de], we1_ref[...],
            preferred_element_type=jnp.float32)
        h = jnp.maximum(agg, 0.0)
        o_ref[...] = (
            jnp.dot(h, w2_ref[...], preferred_element_type=jnp.float32)
            + b2_ref[...]
        )

    return pl.pallas_call(
        body,
        grid=(n // rb,),
        in_specs=[
            pl.BlockSpec((2, rb, d), lambda i: (0, i, 0)),
            pl.BlockSpec((2, rb, 128), lambda i: (0, i, 0)),
            pl.BlockSpec((de, d), lambda i: (0, 0)),
            pl.BlockSpec((d, d), lambda i: (0, 0)),
            pl.BlockSpec((1, d), lambda i: (0, 0)),
        ],
        out_specs=pl.BlockSpec((rb, d), lambda i: (i, 0)),
        out_shape=jax.ShapeDtypeStruct((n, d), jnp.float32),
    )(p1, ea, we1, w2, b2)


def _tc_final(p2, ea, we2, batch3, wlin, blin, n, nb):
    d = p2.shape[2]
    de = we2.shape[0]
    c = wlin.shape[1]
    rb = n // nb

    def body(p_ref, ea_ref, we2_ref, b_ref, wlin_ref, blin_ref, o_ref, acc):
        i = pl.program_id(0)

        @pl.when(i == 0)
        def _init():
            acc[...] = jnp.zeros_like(acc)

        hf = p_ref[0] + p_ref[1] + jnp.dot(
            ea_ref[0, :, :de] + ea_ref[1, :, :de], we2_ref[...],
            preferred_element_type=jnp.float32)
        bv = b_ref[0, 0, :]
        onehot = (bv[None, :] ==
                  lax.broadcasted_iota(jnp.int32, (_B, rb), 0)
                  ).astype(jnp.float32)
        acc[...] += jnp.dot(onehot, hf, preferred_element_type=jnp.float32)

        @pl.when(i == nb - 1)
        def _fin():
            o_ref[...] = (
                jnp.dot(acc[...], wlin_ref[...],
                        preferred_element_type=jnp.float32)
                + blin_ref[...]
            )

    return pl.pallas_call(
        body,
        grid=(nb,),
        in_specs=[
            pl.BlockSpec((2, rb, d), lambda i: (0, i, 0)),
            pl.BlockSpec((2, rb, 128), lambda i: (0, i, 0)),
            pl.BlockSpec((de, d), lambda i: (0, 0)),
            pl.BlockSpec((1, 1, rb), lambda i: (i, 0, 0)),
            pl.BlockSpec((d, c), lambda i: (0, 0)),
            pl.BlockSpec((1, c), lambda i: (0, 0)),
        ],
        out_specs=pl.BlockSpec((_B, c), lambda i: (0, 0)),
        out_shape=jax.ShapeDtypeStruct((_B, c), jnp.float32),
        scratch_shapes=[pltpu.VMEM((_B, d), jnp.float32)],
    )(p2, ea, we2, batch3, wlin, blin)


def kernel(x, edge_index, edge_attr, batch, W1, b1, We1, W2, b2, We2,
           Wlin, blin):
    n, d = x.shape
    nb = 10
    n_pad = 10240

    e = edge_index.shape[1]
    de = edge_attr.shape[1]
    nseg, seg = 5, 25
    nch = e // (_NW * _CH)
    ei5 = edge_index.reshape(2, _NW, nseg, seg, _CH)
    ea4 = edge_attr.reshape(_NW, nch, (_CH * de) // 128, 128)

    ea_p = _sc_ea_pass(ei5, ea4, n_pad, de)
    h1 = _tc_linear(x, W1, b1.reshape(1, -1), n // 5)
    p1 = _sc_gather_pass(h1, ei5, n_pad)
    h2 = _tc_mid(p1, ea_p, We1, W2, b2.reshape(1, -1), n, n // 5)
    p2 = _sc_gather_pass(h2, ei5, n_pad)
    return _tc_final(p2, ea_p, We2, batch.reshape(nb, 1, -1),
                     Wlin, blin.reshape(1, -1), n, nb)
